# baseline (speedup 1.0000x reference)
import functools

import jax
import jax.numpy as jnp
from jax import lax
from jax.experimental import pallas as pl
from jax.experimental.pallas import tpu as pltpu
from jax.experimental.pallas import tpu_sc as plsc

N = 10000
E = 160000
D = 256
H = 8
DH = 32
HALF = 128
NTILES = 16
EPT = E // NTILES
CH = 40
NCHUNK = EPT // CH
NP = 10240
NROWS_PT = NP // NTILES
NZ = NROWS_PT // CH
NB2 = NP // 8
B2PT = NB2 // NTILES
NZ2 = B2PT // CH
INV_SQRT_DH = 1.0 / (DH ** 0.5)

_mesh = plsc.VectorSubcoreMesh(core_axis_name="c", subcore_axis_name="s")

_IDX = lambda: pltpu.VMEM((CH,), jnp.int32)
_ROWS = lambda: pltpu.VMEM((CH, HALF), jnp.float32)


@functools.partial(
    pl.kernel,
    mesh=_mesh,
    out_type=(
        jax.ShapeDtypeStruct((2, NP, HALF), jnp.float32),
        jax.ShapeDtypeStruct((2, NB2, HALF), jnp.float32),
    ),
    scratch_types=[
        pltpu.VMEM((48,), jnp.int32),
        pltpu.VMEM((48,), jnp.int32),
        pltpu.VMEM((CH,), jnp.int32),
        pltpu.VMEM((CH,), jnp.int32),
        pltpu.VMEM((48,), jnp.int32),
        pltpu.VMEM((48,), jnp.int32),
        pltpu.VMEM((CH,), jnp.int32),
        pltpu.VMEM((CH,), jnp.int32),
        pltpu.VMEM((CH, HALF), jnp.float32),
        pltpu.VMEM((CH, HALF), jnp.float32),
        pltpu.VMEM((CH, HALF), jnp.float32),
        pltpu.VMEM((CH, HALF), jnp.float32),
        pltpu.VMEM((CH, HALF), jnp.float32),
        pltpu.VMEM((CH, HALF), jnp.float32),
        pltpu.VMEM_SHARED((NP, HALF), jnp.float32),
        pltpu.VMEM_SHARED((NB2, HALF), jnp.float32),
        pltpu.SemaphoreType.DMA,
        pltpu.SemaphoreType.DMA,
        pltpu.SemaphoreType.DMA,
        pltpu.SemaphoreType.DMA,
        pltpu.SemaphoreType.DMA,
        pltpu.SemaphoreType.DMA,
    ],
)
def _edge_kernel(src_hbm, dst_hbm, qcat, kcat, vcat, out_hbm, den_hbm,
                 isrc0, idst0, idstl0, idx20, isrc1, idst1, idstl1, idx21,
                 qr0, kr0, vr0, qr1, kr1, vr1, acc, acc2,
                 sg0, sg1, sa0, sa1, sd0, sd1):
    c = lax.axis_index("c")
    s = lax.axis_index("s")

    sets = (
        dict(isrc=isrc0, idst=idst0, idstl=idstl0, idx2=idx20,
             qr=qr0, kr=kr0, vr=vr0, sg=sg0, sa=sa0, sd=sd0),
        dict(isrc=isrc1, idst=idst1, idstl=idstl1, idx2=idx21,
             qr=qr1, kr=kr1, vr=vr1, sg=sg1, sa=sa1, sd=sd1),
    )

    zeros16 = jnp.zeros((16,), jnp.float32)

    def zrow(i, carry):
        for j in range(HALF // 16):
            kr0[i, pl.ds(j * 16, 16)] = zeros16
        return carry

    lax.fori_loop(0, CH, zrow, 0)
    for z in range(NZ):
        pltpu.sync_copy(kr0, acc.at[pl.ds(s * NROWS_PT + z * CH, CH)])
    for z in range(NZ2):
        pltpu.sync_copy(kr0, acc2.at[pl.ds(s * B2PT + z * CH, CH)])
    plsc.subcore_barrier()

    coff = c * N
    ebase0 = s * EPT
    lane = lax.broadcasted_iota(jnp.int32, (16,), 0)
    perms = [lane ^ k for k in (1, 2, 4, 8)]
    rotp = (lane + 8) & 15
    low8 = lane < 8
    _dnums = lax.GatherDimensionNumbers(
        offset_dims=(), collapsed_slice_dims=(0,), start_index_map=(0,))

    def _vtake(vv, idx):
        return lax.gather(vv, idx[:, None], dimension_numbers=_dnums,
                          slice_sizes=(1,),
                          mode=lax.GatherScatterMode.PROMISE_IN_BOUNDS)

    def allsum(vv):
        for p in perms:
            vv = vv + _vtake(vv, p)
        return vv

    def fetch_idx(cg, b):
        ebase = ebase0 + cg * CH
        pltpu.sync_copy(src_hbm.at[pl.ds(ebase, CH)], b["isrc"].at[pl.ds(0, CH)])
        pltpu.sync_copy(dst_hbm.at[pl.ds(ebase, CH)], b["idst"].at[pl.ds(0, CH)])
        for j in range(3):
            sl = pl.ds(j * 16, 16)
            sv = b["isrc"][sl] + coff
            dv = b["idst"][sl] + coff
            if j == 2:
                sv = jnp.where(low8, sv, 0)
                dv = jnp.where(low8, dv, 0)
            b["isrc"][sl] = sv
            b["idst"][sl] = dv

    def fire_gathers(b):
        pltpu.async_copy(qcat.at[b["isrc"].at[pl.ds(0, CH)]], b["qr"], b["sg"])
        pltpu.async_copy(kcat.at[b["idst"].at[pl.ds(0, CH)]], b["kr"], b["sg"])
        pltpu.async_copy(vcat.at[b["isrc"].at[pl.ds(0, CH)]], b["vr"], b["sg"])

    def wait_gathers(b):
        pltpu.make_async_copy(
            qcat.at[b["isrc"].at[pl.ds(0, CH)]], b["qr"], b["sg"]).wait()
        pltpu.make_async_copy(
            kcat.at[b["idst"].at[pl.ds(0, CH)]], b["kr"], b["sg"]).wait()
        pltpu.make_async_copy(
            vcat.at[b["isrc"].at[pl.ds(0, CH)]], b["vr"], b["sg"]).wait()

    def compute(b):
        qr, kr, vr = b["qr"], b["kr"], b["vr"]
        idstl, idx2 = b["idstl"], b["idx2"]

        def do_edges(j, dv_raw, nedges):
            for e in range(nedges):
                i = j * 16 + e
                prods = []
                for jj in range(8):
                    sl = pl.ds(jj * 16, 16)
                    prods.append(qr[i, sl] * kr[i, sl])
                exvecs = []
                for h in range(4):
                    s2 = prods[2 * h] + prods[2 * h + 1]
                    exvecs.append(jnp.exp(allsum(s2) * INV_SQRT_DH))
                for jj in range(8):
                    sl = pl.ds(jj * 16, 16)
                    qr[i, sl] = vr[i, sl] * exvecs[jj // 2]
                evec = jnp.zeros((16,), jnp.float32)
                for h in range(4):
                    evec = jnp.where(lane == h, exvecs[h], evec)
                for jj in range(8):
                    kr[i, pl.ds(jj * 16, 16)] = zeros16
                off = pl.multiple_of((dv_raw[e] & 7) * 16, 16)
                kr[i, pl.ds(off, 16)] = evec

        def group_body(j, ecarry):
            dv = b["idst"][pl.ds(j * 16, 16)] - coff
            sl = pl.ds(j * 16, 16)
            idstl[sl] = dv
            idx2[sl] = lax.shift_right_logical(dv, 3)
            do_edges(j, dv, 16)
            return ecarry

        lax.fori_loop(0, 2, group_body, 0)
        dv1 = b["idst"][pl.ds(16, 16)] - coff
        dv2 = b["idst"][pl.ds(32, 16)] - coff
        m = jnp.where(low8, _vtake(dv1, rotp), _vtake(dv2, rotp))
        idstl[pl.ds(24, 16)] = m
        idx2[pl.ds(24, 16)] = lax.shift_right_logical(m, 3)
        do_edges(2, dv2, 8)

    def fire_scatters(b):
        cpa = pltpu.async_copy(b["qr"], acc.at[b["idstl"]], b["sa"], add=True)
        cpd = pltpu.async_copy(b["kr"], acc2.at[b["idx2"]], b["sd"], add=True)
        return cpa, cpd

    fetch_idx(0, sets[0])
    fire_gathers(sets[0])
    fetch_idx(1, sets[1])
    fire_gathers(sets[1])

    def pair_body(k2, carry):
        for p in (0, 1):
            b = sets[p]
            cg = k2 * 2 + p
            wait_gathers(b)
            compute(b)
            cpa, cpd = fire_scatters(b)
            fetch_idx(cg + 2, b)
            cpa.wait()
            cpd.wait()
            fire_gathers(b)
        return carry

    lax.fori_loop(0, (NCHUNK - 2) // 2, pair_body, 0)
    for p in (0, 1):
        b = sets[p]
        wait_gathers(b)
        compute(b)
        cpa, cpd = fire_scatters(b)
        cpa.wait()
        cpd.wait()

    plsc.subcore_barrier()
    for z in range(NZ):
        r0 = s * NROWS_PT + z * CH
        pltpu.sync_copy(acc.at[pl.ds(r0, CH)], out_hbm.at[c, pl.ds(r0, CH)])
    for z in range(NZ2):
        b0 = s * B2PT + z * CH
        pltpu.sync_copy(acc2.at[pl.ds(b0, CH)], den_hbm.at[c, pl.ds(b0, CH)])


def _proj_body(x_ref, w_ref, b_ref, o_ref):
    o_ref[...] = jnp.dot(x_ref[...], w_ref[...],
                         preferred_element_type=jnp.float32) + b_ref[...]


def _final_body(o_ref, d_ref, s_ref, w_ref, b_ref, z_ref):
    dfull = jnp.dot(d_ref[...], s_ref[...], preferred_element_type=jnp.float32)
    d = jnp.maximum(dfull, 1e-30)
    z_ref[...] = jnp.dot(o_ref[...] / d, w_ref[...],
                         preferred_element_type=jnp.float32) + b_ref[...]


_MB = 400


def _proj(x, wcat, bcat):
    return pl.pallas_call(
        _proj_body,
        grid=(N // _MB,),
        in_specs=[
            pl.BlockSpec((_MB, D), lambda i: (i, 0)),
            pl.BlockSpec((D, 3 * D), lambda i: (0, 0)),
            pl.BlockSpec((1, 3 * D), lambda i: (0, 0)),
        ],
        out_specs=pl.BlockSpec((_MB, 3 * D), lambda i: (i, 0)),
        out_shape=jax.ShapeDtypeStruct((N, 3 * D), jnp.float32),
    )(x, wcat, bcat)


def _final(o, dflat, sel, wo, bo):
    return pl.pallas_call(
        _final_body,
        grid=(N // _MB,),
        in_specs=[
            pl.BlockSpec((_MB, D), lambda i: (i, 0)),
            pl.BlockSpec((_MB, 32), lambda i: (i, 0)),
            pl.BlockSpec((32, D), lambda i: (0, 0)),
            pl.BlockSpec((D, D), lambda i: (0, 0)),
            pl.BlockSpec((1, D), lambda i: (0, 0)),
        ],
        out_specs=pl.BlockSpec((_MB, D), lambda i: (i, 0)),
        out_shape=jax.ShapeDtypeStruct((N, D), jnp.float32),
    )(o, dflat, sel, wo, bo)


def _halves(a):
    return a.reshape(N, 2, HALF).transpose(1, 0, 2).reshape(2 * N, HALF)


def kernel(x, edge_index, Wq, bq, Wk, bk, Wv, bv, Wo, bo):
    src = edge_index[0].astype(jnp.int32)
    dst = edge_index[1].astype(jnp.int32)
    wcat = jnp.concatenate([Wq, Wk, Wv], axis=1)
    bcat = jnp.concatenate([bq, bk, bv]).reshape(1, 3 * D)
    y = _proj(x, wcat, bcat)
    q = y[:, :D]
    k = y[:, D:2 * D]
    v = y[:, 2 * D:]
    out, dbuckets = _edge_kernel(src, dst, _halves(q), _halves(k), _halves(v))
    o_un = jnp.concatenate([out[0, :N, :], out[1, :N, :]], axis=1)
    dflat = dbuckets.reshape(2, NB2, 8, 16).transpose(1, 2, 0, 3)
    dflat = dflat.reshape(NP, 32)[:N]
    r = jnp.arange(32)
    head = (r // 16) * 4 + (r % 16)
    valid = (r % 16) < 4
    sel = ((jnp.arange(D)[None, :] // DH == head[:, None]) &
           valid[:, None]).astype(jnp.float32)
    return _final(o_un, dflat, sel, Wo, bo.reshape(1, D))


# speedup vs baseline: 1.3395x; 1.3395x over previous
import functools

import jax
import jax.numpy as jnp
from jax import lax
from jax.experimental import pallas as pl
from jax.experimental.pallas import tpu as pltpu
from jax.experimental.pallas import tpu_sc as plsc

N = 10000
E = 160000
D = 256
H = 8
DH = 32
HALF = 128
NTILES = 16
EPT = E // NTILES
CH = 80
SCH = 2000
NSC = EPT // SCH
WPS = SCH // CH
NCHUNK = EPT // CH
NP = 10240
NROWS_PT = NP // NTILES
NZ = NROWS_PT // CH
NB2 = NP // 8
B2PT = NB2 // NTILES
INV_SQRT_DH = 1.0 / (DH ** 0.5)

_mesh = plsc.VectorSubcoreMesh(core_axis_name="c", subcore_axis_name="s")


@functools.partial(
    pl.kernel,
    mesh=_mesh,
    out_type=(
        jax.ShapeDtypeStruct((2, NP, HALF), jnp.float32),
        jax.ShapeDtypeStruct((2, NB2, HALF), jnp.float32),
    ),
    scratch_types=[
        pltpu.VMEM((SCH,), jnp.int32),
        pltpu.VMEM((SCH,), jnp.int32),
        pltpu.VMEM((CH,), jnp.int32),
        pltpu.VMEM((CH,), jnp.int32),
        pltpu.VMEM((CH, HALF), jnp.float32),
        pltpu.VMEM((CH, HALF), jnp.float32),
        pltpu.VMEM((CH, HALF), jnp.float32),
        pltpu.VMEM_SHARED((NP, HALF), jnp.float32),
        pltpu.VMEM_SHARED((NB2, HALF), jnp.float32),
        pltpu.SemaphoreType.DMA,
        pltpu.SemaphoreType.DMA,
        pltpu.SemaphoreType.DMA,
        pltpu.SemaphoreType.DMA,
        pltpu.SemaphoreType.DMA,
    ],
)
def _edge_kernel(src_hbm, dst_hbm, qcat, kcat, vcat, out_hbm, den_hbm,
                 isb, idb, idstl, idx2, qr, kr, vr, acc, acc2,
                 sem_q, sem_k, sem_v, sem_a, sem_d):
    c = lax.axis_index("c")
    s = lax.axis_index("s")

    zeros16 = jnp.zeros((16,), jnp.float32)

    def zrow(i, carry):
        for j in range(HALF // 16):
            qr[i, pl.ds(j * 16, 16)] = zeros16
        return carry

    lax.fori_loop(0, CH, zrow, 0)
    for z in range(NZ):
        pltpu.sync_copy(qr, acc.at[pl.ds(s * NROWS_PT + z * CH, CH)])
    pltpu.sync_copy(qr, acc2.at[pl.ds(s * B2PT, B2PT)])
    plsc.subcore_barrier()

    coff = c * N
    lane = lax.broadcasted_iota(jnp.int32, (16,), 0)
    perms = [lane ^ k for k in (1, 2, 4, 8)]
    _dnums = lax.GatherDimensionNumbers(
        offset_dims=(), collapsed_slice_dims=(0,), start_index_map=(0,))

    def _vtake(vv, idx):
        return lax.gather(vv, idx[:, None], dimension_numbers=_dnums,
                          slice_sizes=(1,),
                          mode=lax.GatherScatterMode.PROMISE_IN_BOUNDS)

    def allsum(vv):
        for p in perms:
            vv = vv + _vtake(vv, p)
        return vv

    def chunk_body(g, carry):
        w0 = g * CH
        cp_q = pltpu.async_copy(qcat.at[isb.at[pl.ds(w0, CH)]], qr, sem_q)
        cp_k = pltpu.async_copy(kcat.at[idb.at[pl.ds(w0, CH)]], kr, sem_k)
        cp_v = pltpu.async_copy(vcat.at[isb.at[pl.ds(w0, CH)]], vr, sem_v)
        cp_q.wait()
        cp_k.wait()
        cp_v.wait()

        def group_body(g2, ecarry):
            sl = pl.ds(g2 * 16, 16)
            dvec16 = idb[pl.ds(w0 + g2 * 16, 16)] - coff
            idstl[sl] = dvec16
            idx2[sl] = lax.shift_right_logical(dvec16, 3)
            for e in range(16):
                i = g2 * 16 + e
                prods = []
                for j in range(8):
                    sl = pl.ds(j * 16, 16)
                    prods.append(qr[i, sl] * kr[i, sl])
                exvecs = []
                for h in range(4):
                    s2 = prods[2 * h] + prods[2 * h + 1]
                    exvecs.append(jnp.exp(allsum(s2) * INV_SQRT_DH))
                for j in range(8):
                    sl = pl.ds(j * 16, 16)
                    qr[i, sl] = vr[i, sl] * exvecs[j // 2]
                evec = jnp.zeros((16,), jnp.float32)
                for h in range(4):
                    evec = jnp.where(lane == h, exvecs[h], evec)
                for j in range(8):
                    kr[i, pl.ds(j * 16, 16)] = zeros16
                off = pl.multiple_of((dvec16[e] & 7) * 16, 16)
                kr[i, pl.ds(off, 16)] = evec
            return ecarry

        lax.fori_loop(0, CH // 16, group_body, 0)
        cpa = pltpu.async_copy(qr, acc.at[idstl], sem_a, add=True)
        cpd = pltpu.async_copy(kr, acc2.at[idx2], sem_d, add=True)
        cpa.wait()
        cpd.wait()
        return carry

    def super_body(u, carry):
        ebase = s * EPT + u * SCH
        pltpu.sync_copy(src_hbm.at[pl.ds(ebase, SCH)], isb)
        pltpu.sync_copy(dst_hbm.at[pl.ds(ebase, SCH)], idb)
        for j in range(SCH // 16):
            sl = pl.ds(j * 16, 16)
            isb[sl] = isb[sl] + coff
            idb[sl] = idb[sl] + coff
        lax.fori_loop(0, WPS, chunk_body, 0)
        return carry

    lax.fori_loop(0, NSC, super_body, 0)

    plsc.subcore_barrier()
    for z in range(NZ):
        r0 = s * NROWS_PT + z * CH
        pltpu.sync_copy(acc.at[pl.ds(r0, CH)], out_hbm.at[c, pl.ds(r0, CH)])
    b0 = s * B2PT
    pltpu.sync_copy(acc2.at[pl.ds(b0, B2PT)], den_hbm.at[c, pl.ds(b0, B2PT)])


def _proj_body(x_ref, w_ref, b_ref, o_ref):
    o_ref[...] = jnp.dot(x_ref[...], w_ref[...],
                         preferred_element_type=jnp.float32) + b_ref[...]


def _final_body(o_ref, d_ref, s_ref, w_ref, b_ref, z_ref):
    dfull = jnp.dot(d_ref[...], s_ref[...], preferred_element_type=jnp.float32)
    d = jnp.maximum(dfull, 1e-30)
    z_ref[...] = jnp.dot(o_ref[...] / d, w_ref[...],
                         preferred_element_type=jnp.float32) + b_ref[...]


_MB = 400


def _proj(x, wcat, bcat):
    return pl.pallas_call(
        _proj_body,
        grid=(N // _MB,),
        in_specs=[
            pl.BlockSpec((_MB, D), lambda i: (i, 0)),
            pl.BlockSpec((D, 3 * D), lambda i: (0, 0)),
            pl.BlockSpec((1, 3 * D), lambda i: (0, 0)),
        ],
        out_specs=pl.BlockSpec((_MB, 3 * D), lambda i: (i, 0)),
        out_shape=jax.ShapeDtypeStruct((N, 3 * D), jnp.float32),
    )(x, wcat, bcat)


def _final(o, dflat, sel, wo, bo):
    return pl.pallas_call(
        _final_body,
        grid=(N // _MB,),
        in_specs=[
            pl.BlockSpec((_MB, D), lambda i: (i, 0)),
            pl.BlockSpec((_MB, 32), lambda i: (i, 0)),
            pl.BlockSpec((32, D), lambda i: (0, 0)),
            pl.BlockSpec((D, D), lambda i: (0, 0)),
            pl.BlockSpec((1, D), lambda i: (0, 0)),
        ],
        out_specs=pl.BlockSpec((_MB, D), lambda i: (i, 0)),
        out_shape=jax.ShapeDtypeStruct((N, D), jnp.float32),
    )(o, dflat, sel, wo, bo)


def _halves(a):
    return a.reshape(N, 2, HALF).transpose(1, 0, 2).reshape(2 * N, HALF)


def kernel(x, edge_index, Wq, bq, Wk, bk, Wv, bv, Wo, bo):
    src = edge_index[0].astype(jnp.int32)
    dst = edge_index[1].astype(jnp.int32)
    wcat = jnp.concatenate([Wq, Wk, Wv], axis=1)
    bcat = jnp.concatenate([bq, bk, bv]).reshape(1, 3 * D)
    y = _proj(x, wcat, bcat)
    q = y[:, :D]
    k = y[:, D:2 * D]
    v = y[:, 2 * D:]
    out, dbuckets = _edge_kernel(src, dst, _halves(q), _halves(k), _halves(v))
    o_un = jnp.concatenate([out[0, :N, :], out[1, :N, :]], axis=1)
    dflat = dbuckets.reshape(2, NB2, 8, 16).transpose(1, 2, 0, 3)
    dflat = dflat.reshape(NP, 32)[:N]
    r = jnp.arange(32)
    head = (r // 16) * 4 + (r % 16)
    valid = (r % 16) < 4
    sel = ((jnp.arange(D)[None, :] // DH == head[:, None]) &
           valid[:, None]).astype(jnp.float32)
    return _final(o_un, dflat, sel, Wo, bo.reshape(1, D))


# speedup vs baseline: 1.4232x; 1.0624x over previous
import functools

import jax
import jax.numpy as jnp
from jax import lax
from jax.experimental import pallas as pl
from jax.experimental.pallas import tpu as pltpu
from jax.experimental.pallas import tpu_sc as plsc

N = 10000
E = 160000
D = 256
H = 8
DH = 32
HALF = 128
NTILES = 16
EPT = E // NTILES
CH = 80
SCH = 2000
NSC = EPT // SCH
WPS = SCH // CH
NCHUNK = EPT // CH
NP = 10240
NROWS_PT = NP // NTILES
NZ = NROWS_PT // CH
NB2 = NP // 8
B2PT = NB2 // NTILES
INV_SQRT_DH = 1.0 / (DH ** 0.5)

_mesh = plsc.VectorSubcoreMesh(core_axis_name="c", subcore_axis_name="s")


@functools.partial(
    pl.kernel,
    mesh=_mesh,
    out_type=(
        jax.ShapeDtypeStruct((2, NP, HALF), jnp.float32),
        jax.ShapeDtypeStruct((2, NB2, HALF), jnp.float32),
    ),
    scratch_types=[
        pltpu.VMEM((SCH,), jnp.int32),
        pltpu.VMEM((SCH,), jnp.int32),
        pltpu.VMEM((CH,), jnp.int32),
        pltpu.VMEM((CH,), jnp.int32),
        pltpu.VMEM((CH, HALF), jnp.float32),
        pltpu.VMEM((CH, HALF), jnp.float32),
        pltpu.VMEM((CH, HALF), jnp.float32),
        pltpu.VMEM_SHARED((NP, HALF), jnp.float32),
        pltpu.VMEM_SHARED((NB2, HALF), jnp.float32),
        pltpu.SemaphoreType.DMA,
        pltpu.SemaphoreType.DMA,
        pltpu.SemaphoreType.DMA,
        pltpu.SemaphoreType.DMA,
        pltpu.SemaphoreType.DMA,
    ],
)
def _edge_kernel(src_hbm, dst_hbm, qcat, kcat, vcat, out_hbm, den_hbm,
                 isb, idb, idstl, idx2, qr, kr, vr, acc, acc2,
                 sem_q, sem_k, sem_v, sem_a, sem_d):
    c = lax.axis_index("c")
    s = lax.axis_index("s")

    zeros16 = jnp.zeros((16,), jnp.float32)

    def zrow(i, carry):
        for j in range(HALF // 16):
            qr[i, pl.ds(j * 16, 16)] = zeros16
        return carry

    lax.fori_loop(0, CH, zrow, 0)
    for z in range(NZ):
        pltpu.sync_copy(qr, acc.at[pl.ds(s * NROWS_PT + z * CH, CH)])
    pltpu.sync_copy(qr, acc2.at[pl.ds(s * B2PT, B2PT)])
    plsc.subcore_barrier()

    coff = c * N
    lane = lax.broadcasted_iota(jnp.int32, (16,), 0)
    perms = [lane ^ k for k in (1, 2, 4, 8)]
    _dnums = lax.GatherDimensionNumbers(
        offset_dims=(), collapsed_slice_dims=(0,), start_index_map=(0,))

    def _vtake(vv, idx):
        return lax.gather(vv, idx[:, None], dimension_numbers=_dnums,
                          slice_sizes=(1,),
                          mode=lax.GatherScatterMode.PROMISE_IN_BOUNDS)

    def allsum(vv):
        for p in perms:
            vv = vv + _vtake(vv, p)
        return vv

    def chunk_body(g, carry):
        w0 = g * CH
        cp_q = pltpu.async_copy(qcat.at[isb.at[pl.ds(w0, CH)]], qr, sem_q)
        cp_k = pltpu.async_copy(kcat.at[idb.at[pl.ds(w0, CH)]], kr, sem_k)
        cp_v = pltpu.async_copy(vcat.at[isb.at[pl.ds(w0, CH)]], vr, sem_v)
        cp_q.wait()
        cp_k.wait()
        cp_v.wait()

        def group_body(g2, ecarry):
            sl = pl.ds(g2 * 16, 16)
            dvec16 = idb[pl.ds(w0 + g2 * 16, 16)] - coff
            idstl[sl] = dvec16
            idx2[sl] = lax.shift_right_logical(dvec16, 3)
            for e in range(16):
                i = g2 * 16 + e
                prods = []
                for j in range(8):
                    sl = pl.ds(j * 16, 16)
                    prods.append(qr[i, sl] * kr[i, sl])
                exvecs = []
                for h in range(4):
                    s2 = prods[2 * h] + prods[2 * h + 1]
                    exvecs.append(jnp.exp(allsum(s2) * INV_SQRT_DH))
                for j in range(8):
                    sl = pl.ds(j * 16, 16)
                    qr[i, sl] = vr[i, sl] * exvecs[j // 2]
                evec = jnp.zeros((16,), jnp.float32)
                for h in range(4):
                    evec = jnp.where(lane == h, exvecs[h], evec)
                for j in range(8):
                    kr[i, pl.ds(j * 16, 16)] = zeros16
                off = pl.multiple_of((dvec16[e] & 7) * 16, 16)
                kr[i, pl.ds(off, 16)] = evec
            return ecarry

        lax.fori_loop(0, CH // 16, group_body, 0)
        cpa = pltpu.async_copy(qr, acc.at[idstl], sem_a, add=True)
        cpd = pltpu.async_copy(kr, acc2.at[idx2], sem_d, add=True)
        cpa.wait()
        cpd.wait()
        return carry

    def super_body(u, carry):
        ebase = s * EPT + u * SCH
        pltpu.sync_copy(src_hbm.at[pl.ds(ebase, SCH)], isb)
        pltpu.sync_copy(dst_hbm.at[pl.ds(ebase, SCH)], idb)
        for j in range(SCH // 16):
            sl = pl.ds(j * 16, 16)
            isb[sl] = isb[sl] + coff
            idb[sl] = idb[sl] + coff
        lax.fori_loop(0, WPS, chunk_body, 0)
        return carry

    lax.fori_loop(0, NSC, super_body, 0)

    plsc.subcore_barrier()
    for z in range(NZ):
        r0 = s * NROWS_PT + z * CH
        pltpu.sync_copy(acc.at[pl.ds(r0, CH)], out_hbm.at[c, pl.ds(r0, CH)])
    b0 = s * B2PT
    pltpu.sync_copy(acc2.at[pl.ds(b0, B2PT)], den_hbm.at[c, pl.ds(b0, B2PT)])


def _proj_body(x_ref, w_ref, b_ref, q_ref, k_ref, v_ref):
    y = jnp.dot(x_ref[...], w_ref[...],
                preferred_element_type=jnp.float32) + b_ref[...]
    q_ref[0] = y[:, 0 * HALF:1 * HALF]
    q_ref[1] = y[:, 1 * HALF:2 * HALF]
    k_ref[0] = y[:, 2 * HALF:3 * HALF]
    k_ref[1] = y[:, 3 * HALF:4 * HALF]
    v_ref[0] = y[:, 4 * HALF:5 * HALF]
    v_ref[1] = y[:, 5 * HALF:6 * HALF]


def _final_body(o_ref, d_ref, s_ref, w_ref, b_ref, z_ref):
    o = jnp.concatenate([o_ref[0], o_ref[1]], axis=1)
    dfull = jnp.dot(d_ref[...], s_ref[...], preferred_element_type=jnp.float32)
    d = jnp.maximum(dfull, 1e-30)
    z_ref[...] = jnp.dot(o / d, w_ref[...],
                         preferred_element_type=jnp.float32) + b_ref[...]


_MB = 400


def _proj(x, wcat, bcat):
    spec = pl.BlockSpec((2, _MB, HALF), lambda i: (0, i, 0))
    shp = jax.ShapeDtypeStruct((2, N, HALF), jnp.float32)
    return pl.pallas_call(
        _proj_body,
        grid=(N // _MB,),
        in_specs=[
            pl.BlockSpec((_MB, D), lambda i: (i, 0)),
            pl.BlockSpec((D, 3 * D), lambda i: (0, 0)),
            pl.BlockSpec((1, 3 * D), lambda i: (0, 0)),
        ],
        out_specs=[spec, spec, spec],
        out_shape=[shp, shp, shp],
    )(x, wcat, bcat)


def _final(o, dflat, sel, wo, bo):
    return pl.pallas_call(
        _final_body,
        grid=(N // _MB,),
        in_specs=[
            pl.BlockSpec((2, _MB, HALF), lambda i: (0, i, 0)),
            pl.BlockSpec((_MB, 32), lambda i: (i, 0)),
            pl.BlockSpec((32, D), lambda i: (0, 0)),
            pl.BlockSpec((D, D), lambda i: (0, 0)),
            pl.BlockSpec((1, D), lambda i: (0, 0)),
        ],
        out_specs=pl.BlockSpec((_MB, D), lambda i: (i, 0)),
        out_shape=jax.ShapeDtypeStruct((N, D), jnp.float32),
    )(o, dflat, sel, wo, bo)


def kernel(x, edge_index, Wq, bq, Wk, bk, Wv, bv, Wo, bo):
    src = edge_index[0].astype(jnp.int32)
    dst = edge_index[1].astype(jnp.int32)
    wcat = jnp.concatenate([Wq, Wk, Wv], axis=1)
    bcat = jnp.concatenate([bq, bk, bv]).reshape(1, 3 * D)
    q3, k3, v3 = _proj(x, wcat, bcat)
    out, dbuckets = _edge_kernel(src, dst,
                                 q3.reshape(2 * N, HALF),
                                 k3.reshape(2 * N, HALF),
                                 v3.reshape(2 * N, HALF))
    dflat = dbuckets.reshape(2, NB2, 8, 16).transpose(1, 2, 0, 3)
    dflat = dflat.reshape(NP, 32)[:N]
    r = jnp.arange(32)
    head = (r // 16) * 4 + (r % 16)
    valid = (r % 16) < 4
    sel = ((jnp.arange(D)[None, :] // DH == head[:, None]) &
           valid[:, None]).astype(jnp.float32)
    return _final(out, dflat, sel, Wo, bo.reshape(1, D))
